# Initial kernel scaffold; baseline (speedup 1.0000x reference)
import jax
import jax.numpy as jnp
from jax.experimental import pallas as pl


def kernel(x, W1, b1, W2, b2, W3, b3, Wg, bg, Wm1, bm1, Wm2, bm2):
    raise NotImplementedError("write your pallas kernel here")


# speedup vs baseline: 7.4601x; 7.4601x over previous
import numpy as np
import jax
import jax.numpy as jnp
from jax.experimental import pallas as pl

_N = 360


def _build_consts():
    angles = np.linspace(0.0, 2.0 * np.pi, _N)
    cos = np.cos(angles).astype(np.float32).reshape(_N, 1)
    sin = np.sin(angles).astype(np.float32).reshape(_N, 1)
    deg = np.full(_N, 3.0, np.float64)
    deg[0] = deg[-1] = 2.0
    dis = 1.0 / np.sqrt(deg)
    a = np.zeros((_N, _N), np.float64)
    i = np.arange(_N)
    a[i, i] = dis * dis
    a[i[1:], i[:-1]] = dis[1:] * dis[:-1]
    a[i[:-1], i[1:]] = dis[:-1] * dis[1:]
    return cos, sin, a.astype(np.float32)


_COS, _SIN, _A = _build_consts()


def _fused(scan_ref, cos_ref, sin_ref, a_ref,
           w1t_ref, b1_ref, w2t_ref, b2_ref, w3t_ref, b3_ref,
           wgt_ref, bg_ref, wm1t_ref, bm1_ref, wm2t_ref, bm2_ref,
           out_ref):
    f32 = jnp.float32
    hi = jax.lax.Precision.HIGHEST
    scan = scan_ref[:]
    nx = scan * cos_ref[:]
    ny = scan * sin_ref[:]
    a = a_ref[:]

    w1t = w1t_ref[:]
    xw = nx * w1t[0:1, :] + ny * w1t[1:2, :]
    h = jnp.maximum(jnp.dot(a, xw, preferred_element_type=f32, precision=hi) + b1_ref[:], 0.0)

    hw = jnp.dot(h, w2t_ref[:], preferred_element_type=f32, precision=hi)
    h = jnp.maximum(jnp.dot(a, hw, preferred_element_type=f32, precision=hi) + b2_ref[:], 0.0)
    hw = jnp.dot(h, w3t_ref[:], preferred_element_type=f32, precision=hi)
    h = jnp.maximum(jnp.dot(a, hw, preferred_element_type=f32, precision=hi) + b3_ref[:], 0.0)

    g = jnp.mean(h, axis=0, keepdims=True)
    c = jnp.dot(g, wgt_ref[:], preferred_element_type=f32, precision=hi) + bg_ref[:]
    m = jnp.maximum(
        jnp.dot(c, wm1t_ref[:], preferred_element_type=f32, precision=hi) + bm1_ref[:], 0.0)
    out_ref[:] = jnp.dot(m, wm2t_ref[:], preferred_element_type=f32, precision=hi) + bm2_ref[:]


@jax.jit
def _run(x, W1, b1, W2, b2, W3, b3, Wg, bg, Wm1, bm1, Wm2, bm2):
    scan = x[0, :_N].reshape(_N, 1)
    out = pl.pallas_call(
        _fused,
        out_shape=jax.ShapeDtypeStruct((1, 200), jnp.float32),
    )(
        scan, jnp.asarray(_COS), jnp.asarray(_SIN), jnp.asarray(_A),
        W1.T, b1.reshape(1, -1), W2.T, b2.reshape(1, -1),
        W3.T, b3.reshape(1, -1), Wg.T, bg.reshape(1, -1),
        Wm1.T, bm1.reshape(1, -1), Wm2.T, bm2.reshape(1, -1),
    )
    return out.reshape(1, 10, 10, 2)


def kernel(x, W1, b1, W2, b2, W3, b3, Wg, bg, Wm1, bm1, Wm2, bm2):
    return _run(x, W1, b1, W2, b2, W3, b3, Wg, bg, Wm1, bm1, Wm2, bm2)


# speedup vs baseline: 10.7449x; 1.4403x over previous
import numpy as np
import jax
import jax.numpy as jnp
from jax.experimental import pallas as pl

_N = 360


def _build_consts():
    angles = np.linspace(0.0, 2.0 * np.pi, _N)
    cos = np.cos(angles).astype(np.float32).reshape(_N, 1)
    sin = np.sin(angles).astype(np.float32).reshape(_N, 1)
    deg = np.full(_N, 3.0, np.float64)
    deg[0] = deg[-1] = 2.0
    dis = 1.0 / np.sqrt(deg)
    cd = (dis * dis).astype(np.float32).reshape(_N, 1)
    cl = np.zeros((_N, 1), np.float32)
    cl[1:, 0] = (dis[1:] * dis[:-1]).astype(np.float32)
    cu = np.zeros((_N, 1), np.float32)
    cu[:-1, 0] = (dis[:-1] * dis[1:]).astype(np.float32)
    return cos, sin, cl, cd, cu


_COS, _SIN, _CL, _CD, _CU = _build_consts()

_DN_T = (((1,), (1,)), ((), ()))


def _fused(scan_ref, cos_ref, sin_ref, cl_ref, cd_ref, cu_ref,
           w1t_ref, b1_ref, w2_ref, b2_ref, w3_ref, b3_ref,
           wg_ref, bg_ref, wm1_ref, bm1_ref, wm2_ref, bm2_ref,
           out_ref):
    f32 = jnp.float32
    hi = jax.lax.Precision.HIGHEST
    cl, cd, cu = cl_ref[:], cd_ref[:], cu_ref[:]

    def agg(v):
        return cd * v + cl * jnp.roll(v, 1, axis=0) + cu * jnp.roll(v, -1, axis=0)

    def mm_t(v, w_ref):
        return jax.lax.dot_general(v, w_ref[:], _DN_T,
                                   preferred_element_type=f32, precision=hi)

    scan = scan_ref[:]
    nx = scan * cos_ref[:]
    ny = scan * sin_ref[:]

    w1t = w1t_ref[:]
    xw = nx * w1t[0:1, :] + ny * w1t[1:2, :]
    h = jnp.maximum(agg(xw) + b1_ref[:], 0.0)

    h = jnp.maximum(agg(mm_t(h, w2_ref)) + b2_ref[:], 0.0)
    h = jnp.maximum(agg(mm_t(h, w3_ref)) + b3_ref[:], 0.0)

    g = jnp.mean(h, axis=0, keepdims=True)
    c = mm_t(g, wg_ref) + bg_ref[:]
    m = jnp.maximum(mm_t(c, wm1_ref) + bm1_ref[:], 0.0)
    out_ref[:] = mm_t(m, wm2_ref) + bm2_ref[:]


@jax.jit
def _run(x, W1, b1, W2, b2, W3, b3, Wg, bg, Wm1, bm1, Wm2, bm2):
    scan = x[0, :_N].reshape(_N, 1)
    out = pl.pallas_call(
        _fused,
        out_shape=jax.ShapeDtypeStruct((1, 200), jnp.float32),
    )(
        scan, jnp.asarray(_COS), jnp.asarray(_SIN),
        jnp.asarray(_CL), jnp.asarray(_CD), jnp.asarray(_CU),
        W1.T, b1.reshape(1, -1), W2, b2.reshape(1, -1),
        W3, b3.reshape(1, -1), Wg, bg.reshape(1, -1),
        Wm1, bm1.reshape(1, -1), Wm2, bm2.reshape(1, -1),
    )
    return out.reshape(1, 10, 10, 2)


def kernel(x, W1, b1, W2, b2, W3, b3, Wg, bg, Wm1, bm1, Wm2, bm2):
    return _run(x, W1, b1, W2, b2, W3, b3, Wg, bg, Wm1, bm1, Wm2, bm2)
